# baseline (speedup 1.0000x reference)
import functools

import jax
import jax.numpy as jnp
from jax import lax
from jax.experimental import pallas as pl
from jax.experimental.pallas import tpu as pltpu
from jax.experimental.pallas import tpu_sc as plsc

BATCH = 16384
EMB_DIM = 64
NC = 2
NS = 16
NW = NC * NS
B_PER_W = BATCH // NW
GCHUNK = 128
N_GCHUNK = B_PER_W // GCHUNK
L = 16
ROWBLK = 16


def _tower_kernel(uid_hbm, iid_hbm, uemb_hbm, iemb_hbm, out_hbm,
                  idx_u, idx_i, rows_u, rows_i, out_v, sem):
    wid = lax.axis_index("s") * NC + lax.axis_index("c")
    base = wid * B_PER_W

    pltpu.sync_copy(uid_hbm.at[pl.ds(base, B_PER_W)], idx_u)
    pltpu.sync_copy(iid_hbm.at[pl.ds(base, B_PER_W)], idx_i)

    handles = []
    for j in range(N_GCHUNK):
        sl = pl.ds(j * GCHUNK, GCHUNK)
        handles.append(pltpu.async_copy(uemb_hbm.at[idx_u.at[sl]], rows_u.at[sl], sem))
        handles.append(pltpu.async_copy(iemb_hbm.at[idx_i.at[sl]], rows_i.at[sl], sem))
    for h in handles:
        h.wait()

    lane = lax.iota(jnp.int32, L)
    perms = [(jnp.arange(L, dtype=jnp.int32) ^ sh) for sh in (8, 4, 2, 1)]
    _dnums = lax.GatherDimensionNumbers(
        offset_dims=(), collapsed_slice_dims=(0,), start_index_map=(0,))

    def _lane_perm(x, p):
        return lax.gather(x, p[:, None], _dnums, slice_sizes=(1,),
                          mode=lax.GatherScatterMode.PROMISE_IN_BOUNDS)

    def block_body(blk, _):
        r0 = blk * ROWBLK
        acc = jnp.zeros((L,), jnp.float32)
        for r in range(ROWBLK):
            t = jnp.zeros((L,), jnp.float32)
            for k in range(EMB_DIM // L):
                u = rows_u[r0 + r, pl.ds(k * L, L)]
                v = rows_i[r0 + r, pl.ds(k * L, L)]
                t = t + u * v
            for p in perms:
                t = t + _lane_perm(t, p)
            acc = jnp.where(lane == r, t, acc)
        acc = 1.0 / (1.0 + jnp.exp(-acc))
        out_v[pl.ds(blk * ROWBLK, ROWBLK)] = acc
        return _

    lax.fori_loop(0, B_PER_W // ROWBLK, block_body, 0, unroll=False)

    pltpu.sync_copy(out_v, out_hbm.at[pl.ds(base, B_PER_W)])


@functools.partial(jax.jit, static_argnames=())
def kernel(user_ids, item_ids, user_emb, item_emb):
    mesh = plsc.VectorSubcoreMesh(core_axis_name="c", subcore_axis_name="s")
    f = pl.kernel(
        _tower_kernel,
        out_type=jax.ShapeDtypeStruct((BATCH,), jnp.float32),
        mesh=mesh,
        compiler_params=pltpu.CompilerParams(use_tc_tiling_on_sc=False),
        scratch_types=[
            pltpu.VMEM((B_PER_W,), jnp.int32),
            pltpu.VMEM((B_PER_W,), jnp.int32),
            pltpu.VMEM((B_PER_W, EMB_DIM), jnp.float32),
            pltpu.VMEM((B_PER_W, EMB_DIM), jnp.float32),
            pltpu.VMEM((B_PER_W,), jnp.float32),
            pltpu.SemaphoreType.DMA,
        ],
    )
    return f(user_ids.astype(jnp.int32), item_ids.astype(jnp.int32),
             user_emb, item_emb)
